# baseline (speedup 1.0000x reference)
import functools

import jax
import jax.numpy as jnp
from jax import lax
from jax.experimental import pallas as pl
from jax.experimental.pallas import tpu as pltpu
from jax.experimental.pallas import tpu_sc as plsc

EMB = 128
NTAB = 9
ROWS_PER_TILE = 3200
CHUNK_ROWS = 80
BLK = 16


def _sc_geometry():
    try:
        info = plsc.get_sparse_core_info()
        return info.num_cores, info.num_subcores
    except Exception:
        return 2, 16


def kernel(x, T0, T1, T2, T3, T4, T5, T6, T7, T8):
    tables = (T0, T1, T2, T3, T4, T5, T6, T7, T8)
    n = x.shape[0]
    num_cores, num_subcores = _sc_geometry()
    mesh = plsc.VectorSubcoreMesh(core_axis_name="c", subcore_axis_name="s")

    dims = [t.shape[0] for t in tables]
    offs = [0] * NTAB
    for i in range(1, NTAB):
        offs[i] = offs[i - 1] + dims[i - 1]
    total_rows = offs[-1] + dims[-1]

    xflat = x.T.reshape(-1)
    tflat = jnp.concatenate([t.reshape(-1) for t in tables])

    scratch = [
        pltpu.VMEM((total_rows * EMB,), jnp.float32),
        pltpu.VMEM((NTAB * ROWS_PER_TILE,), jnp.int32),
        pltpu.VMEM((CHUNK_ROWS * EMB,), jnp.float32),
    ]

    @functools.partial(
        pl.kernel,
        mesh=mesh,
        out_type=jax.ShapeDtypeStruct((n * EMB,), jnp.float32),
        scratch_types=scratch,
        compiler_params=pltpu.CompilerParams(
            needs_layout_passes=False, use_tc_tiling_on_sc=False
        ),
    )
    def run(x_hbm, t_hbm, out_hbm, tab, xbuf, obuf):
        wid = lax.axis_index("s") * num_cores + lax.axis_index("c")

        pltpu.sync_copy(t_hbm, tab)

        start = wid * ROWS_PER_TILE
        base = jnp.minimum(start, n - ROWS_PER_TILE)
        loff = start - base
        nch = jnp.clip((n - start) // CHUNK_ROWS, 0, ROWS_PER_TILE // CHUNK_ROWS)
        base = pl.multiple_of(base, CHUNK_ROWS)
        for t in range(NTAB):
            pltpu.sync_copy(x_hbm.at[pl.ds(t * n + base, ROWS_PER_TILE)],
                            xbuf.at[pl.ds(t * ROWS_PER_TILE, ROWS_PER_TILE)])

        iota128 = lax.iota(jnp.int32, 16) * EMB

        def chunk_body(ci, carry):
            r0 = loff + ci * CHUNK_ROWS

            def blk_body(bi, carry2):
                r = r0 + bi * BLK
                rowptr = [
                    (xbuf[pl.ds(t * ROWS_PER_TILE + r, BLK)] + offs[t]) * EMB
                    for t in range(NTAB)
                ]
                obase = bi * (BLK * EMB) + iota128

                def col_body(c, carry3):
                    colv = jnp.full((16,), c, jnp.int32)
                    acc = plsc.load_gather(tab, [rowptr[0] + colv])
                    for t in range(1, NTAB):
                        acc = acc + plsc.load_gather(tab, [rowptr[t] + colv])
                    plsc.store_scatter(obuf, [obase + colv], acc)
                    return carry3

                lax.fori_loop(0, EMB, col_body, 0, unroll=8)
                return carry2

            lax.fori_loop(0, CHUNK_ROWS // BLK, blk_body, 0)
            row = pl.multiple_of((start + ci * CHUNK_ROWS) * EMB, CHUNK_ROWS * EMB)
            pltpu.sync_copy(obuf, out_hbm.at[pl.ds(row, CHUNK_ROWS * EMB)])
            return carry

        lax.fori_loop(0, nch, chunk_body, 0)

    return run(xflat, tflat).reshape(n, EMB)


# speedup vs baseline: 46.1240x; 46.1240x over previous
import functools

import jax
import jax.numpy as jnp
from jax import lax
from jax.experimental import pallas as pl
from jax.experimental.pallas import tpu as pltpu
from jax.experimental.pallas import tpu_sc as plsc

EMB = 128
NTAB = 9
ROWS_PER_TILE = 3200
CHUNK_ROWS = 80
BLK = 16


def _sc_geometry():
    try:
        info = plsc.get_sparse_core_info()
        return info.num_cores, info.num_subcores
    except Exception:
        return 2, 16


def kernel(x, T0, T1, T2, T3, T4, T5, T6, T7, T8):
    tables = (T0, T1, T2, T3, T4, T5, T6, T7, T8)
    n = x.shape[0]
    num_cores, num_subcores = _sc_geometry()
    mesh = plsc.VectorSubcoreMesh(core_axis_name="c", subcore_axis_name="s")

    dims = [t.shape[0] for t in tables]
    offs = [0] * NTAB
    for i in range(1, NTAB):
        offs[i] = offs[i - 1] + dims[i - 1]
    total_rows = offs[-1] + dims[-1]

    xflat = x.T.reshape(-1)
    tflat = jnp.concatenate([t.reshape(-1) for t in tables])

    scratch = [
        pltpu.VMEM((total_rows * EMB,), jnp.float32),
        pltpu.VMEM((NTAB * ROWS_PER_TILE,), jnp.int32),
        pltpu.VMEM((CHUNK_ROWS * EMB,), jnp.float32),
    ]

    @functools.partial(
        pl.kernel,
        mesh=mesh,
        out_type=jax.ShapeDtypeStruct((n * EMB,), jnp.float32),
        scratch_types=scratch,
        compiler_params=pltpu.CompilerParams(
            needs_layout_passes=False, use_tc_tiling_on_sc=False
        ),
    )
    def run(x_hbm, t_hbm, out_hbm, tab, xbuf, obuf):
        wid = lax.axis_index("s") * num_cores + lax.axis_index("c")

        pltpu.sync_copy(t_hbm, tab)

        start = wid * ROWS_PER_TILE
        base = jnp.minimum(start, n - ROWS_PER_TILE)
        loff = start - base
        nch = jnp.clip((n - start) // CHUNK_ROWS, 0, ROWS_PER_TILE // CHUNK_ROWS)
        base = pl.multiple_of(base, CHUNK_ROWS)
        for t in range(NTAB):
            pltpu.sync_copy(x_hbm.at[pl.ds(t * n + base, ROWS_PER_TILE)],
                            xbuf.at[pl.ds(t * ROWS_PER_TILE, ROWS_PER_TILE)])

        iota128 = lax.iota(jnp.int32, 16) * EMB

        def chunk_body(ci, carry):
            r0 = loff + ci * CHUNK_ROWS

            def blk_body(bi, carry2):
                r = r0 + bi * BLK
                rowptr = [
                    (xbuf[pl.ds(t * ROWS_PER_TILE + r, BLK)] + offs[t]) * EMB
                    for t in range(NTAB)
                ]
                obase = bi * (BLK * EMB) + iota128

                @functools.partial(plsc.parallel_loop, 0, EMB, unroll=8)
                def col_body(c):
                    colv = jnp.full((16,), c, jnp.int32)
                    vals = [
                        plsc.load_gather(tab, [rowptr[t] + colv])
                        for t in range(NTAB)
                    ]
                    while len(vals) > 1:
                        vals = [
                            vals[i] + vals[i + 1] if i + 1 < len(vals) else vals[i]
                            for i in range(0, len(vals), 2)
                        ]
                    plsc.store_scatter(obuf, [obase + colv], vals[0])

                return carry2

            lax.fori_loop(0, CHUNK_ROWS // BLK, blk_body, 0)
            row = pl.multiple_of((start + ci * CHUNK_ROWS) * EMB, CHUNK_ROWS * EMB)
            pltpu.sync_copy(obuf, out_hbm.at[pl.ds(row, CHUNK_ROWS * EMB)])
            return carry

        lax.fori_loop(0, nch, chunk_body, 0)

    return run(xflat, tflat).reshape(n, EMB)
